# baseline (speedup 1.0000x reference)
import functools

import jax
import jax.numpy as jnp
from jax import lax
from jax.experimental import pallas as pl
from jax.experimental.pallas import tpu as pltpu
from jax.experimental.pallas import tpu_sc as plsc

T, DIM, E, HID = 2048, 768, 8, 1024
TBLK = 256
NTB = T // TBLK
R = 1024
NBLK = T * 2 // R + E
PMAX = NBLK * R

NW = 32
TOK_W = T // NW
CH = 32


HALF = DIM // 2


def _pack_bf16(v):
    u = jax.lax.bitcast_convert_type(v, jnp.uint32)

    def rne(w):
        return (w + jnp.uint32(0x7FFF) + ((w >> 16) & jnp.uint32(1))) & jnp.uint32(0xFFFF0000)

    packed = rne(u[..., HALF:]) | (rne(u[..., :HALF]) >> 16)
    return jax.lax.bitcast_convert_type(packed, jnp.int32)


def _unpack_bf16(p):
    u = jax.lax.bitcast_convert_type(p, jnp.uint32)
    lo = jax.lax.bitcast_convert_type(u << 16, jnp.float32)
    hi = jax.lax.bitcast_convert_type(u & jnp.uint32(0xFFFF0000), jnp.float32)
    return jnp.concatenate([lo, hi], axis=-1)


def _shift_down(a, s):
    return jnp.concatenate([jnp.zeros((s,) + a.shape[1:], a.dtype), a[:-s]], axis=0)


def _shift_right_lanes(a, s):
    return jnp.concatenate([jnp.zeros(a.shape[:-1] + (s,), a.dtype), a[..., :-s]], axis=-1)


def _rm_body(x_ref, wr_ref,
             w01_ref, pos_ref, blk_ref, xbf_ref,
             sidx, hist, carry):
    ph = pl.program_id(0)
    tb = pl.program_id(1)
    lane = jax.lax.broadcasted_iota(jnp.int32, (TBLK, 128), 1)
    lane1 = jax.lax.broadcasted_iota(jnp.int32, (1, 128), 1)
    row = jax.lax.broadcasted_iota(jnp.int32, (8, TBLK), 0)

    @pl.when(ph == 0)
    def _():
        @pl.when(tb == 0)
        def _():
            hist[...] = jnp.zeros_like(hist)
            carry[...] = jnp.zeros_like(carry)

        xbf_ref[...] = _pack_bf16(x_ref[...])
        logits = jnp.dot(x_ref[...], wr_ref[...], preferred_element_type=jnp.float32)
        neg = jnp.float32(-1e30)
        l = jnp.where(lane < E, logits, neg)
        m0 = jnp.max(l, axis=-1, keepdims=True)
        i0 = jnp.min(jnp.where(l == m0, lane, jnp.int32(10**9)), axis=-1, keepdims=True)
        l2 = jnp.where(lane == i0, neg, l)
        m1 = jnp.max(l2, axis=-1, keepdims=True)
        i1 = jnp.min(jnp.where(l2 == m1, lane, jnp.int32(10**9)), axis=-1, keepdims=True)
        w0 = jax.nn.sigmoid(m0 - m1)
        i0r = jnp.broadcast_to(i0[:, 0][None, :], (8, TBLK))
        i1r = jnp.broadcast_to(i1[:, 0][None, :], (8, TBLK))
        packed = jnp.where(row == 0, i0r, jnp.where(row == 1, i1r, 0))
        sidx[:, pl.ds(tb * TBLK, TBLK)] = packed
        w0b = jnp.broadcast_to(w0, (TBLK, 16))
        w01_ref[...] = jnp.concatenate([w0b[None], (1.0 - w0b)[None]], axis=0)
        cnt = (lane == i0).astype(jnp.int32) + (lane == i1).astype(jnp.int32)
        hist[...] += jnp.sum(cnt, axis=0, keepdims=True)

    @pl.when(ph == 1)
    def _():
        tot = hist[...]
        sizes = ((tot + (R - 1)) // R) * R
        sizes = jnp.where(lane1 < E, sizes, 0)
        incl = sizes
        for s in (1, 2, 4):
            incl = incl + _shift_right_lanes(incl, s)
        poff = incl - sizes

        @pl.when(tb == 0)
        def _():
            pb = poff // R
            acc = jnp.full((1, 128), -1, jnp.int32)
            for e in range(E):
                pe = jnp.sum(jnp.where(lane1 == e, pb, 0))
                acc = acc + (lane1 >= pe).astype(jnp.int32)
            nused = jnp.sum(jnp.where(lane1 < E, sizes, 0)) // R
            blk_ref[...] = jnp.where(lane1 == 120, nused, acc)[None]

        i0 = sidx[0, pl.ds(tb * TBLK, TBLK)][:, None]
        i1 = sidx[1, pl.ds(tb * TBLK, TBLK)][:, None]
        cnt = (lane == i0).astype(jnp.int32) + (lane == i1).astype(jnp.int32)
        incl = cnt
        for s in (1, 2, 4, 8, 16, 32, 64, 128):
            incl = incl + _shift_down(incl, s)
        ec = incl - cnt
        base = poff + carry[...] + ec
        pos0 = jnp.sum(jnp.where(lane == i0, base, 0), axis=-1)
        pos1 = jnp.sum(jnp.where(lane == i1, base, 0), axis=-1)
        carry[...] += jnp.sum(cnt, axis=0, keepdims=True)
        p0r = jnp.broadcast_to(pos0[None, :], (8, TBLK))
        p1r = jnp.broadcast_to(pos1[None, :], (8, TBLK))
        pos_ref[...] = jnp.where(row == 0, p0r, jnp.where(row == 1, p1r, 0))


def _router_meta(x2, wr_pad):
    return pl.pallas_call(
        _rm_body,
        grid=(2, NTB),
        in_specs=[
            pl.BlockSpec((TBLK, DIM), lambda p, tb: (tb, 0)),
            pl.BlockSpec((DIM, 128), lambda p, tb: (0, 0)),
        ],
        out_specs=[
            pl.BlockSpec((2, TBLK, 16), lambda p, tb: (0, jnp.where(p == 0, tb, NTB - 1), 0)),
            pl.BlockSpec((8, TBLK), lambda p, tb: (0, jnp.where(p == 1, tb, 0))),
            pl.BlockSpec((1, 1, 128), lambda p, tb: (0, 0, 0)),
            pl.BlockSpec((TBLK, HALF), lambda p, tb: (jnp.where(p == 0, tb, NTB - 1), 0)),
        ],
        out_shape=[
            jax.ShapeDtypeStruct((2, T, 16), jnp.float32),
            jax.ShapeDtypeStruct((8, T), jnp.int32),
            jax.ShapeDtypeStruct((1, 1, 128), jnp.int32),
            jax.ShapeDtypeStruct((T, HALF), jnp.int32),
        ],
        scratch_shapes=[
            pltpu.VMEM((8, T), jnp.int32),
            pltpu.VMEM((1, 128), jnp.int32),
            pltpu.VMEM((1, 128), jnp.int32),
        ],
    )(x2, wr_pad)


def _shared_body(x_ref, w1s_ref, w3s_ref, w2s_ref, shared_ref):
    x = x_ref[...]
    h = jax.nn.silu(jnp.dot(x, w1s_ref[...], preferred_element_type=jnp.float32))
    g = jnp.dot(x, w3s_ref[...], preferred_element_type=jnp.float32)
    shared_ref[...] = jnp.dot(h * g, w2s_ref[...], preferred_element_type=jnp.float32)


def _shared_expert(x2, w1s, w3s, w2s):
    return pl.pallas_call(
        _shared_body,
        grid=(NTB,),
        in_specs=[
            pl.BlockSpec((TBLK, DIM), lambda tb: (tb, 0)),
            pl.BlockSpec((DIM, HID), lambda tb: (0, 0)),
            pl.BlockSpec((DIM, HID), lambda tb: (0, 0)),
            pl.BlockSpec((HID, DIM), lambda tb: (0, 0)),
        ],
        out_specs=pl.BlockSpec((TBLK, DIM), lambda tb: (tb, 0)),
        out_shape=jax.ShapeDtypeStruct((T, DIM), jnp.float32),
    )(x2, w1s, w3s, w2s)


def _dispatch_body(x_hbm, pos_hbm, xs_hbm, p0v, p1v, xbuf, sem):
    wid = lax.axis_index("s") * 2 + lax.axis_index("c")
    base = wid * TOK_W
    pltpu.sync_copy(pos_hbm.at[0, pl.ds(base, TOK_W)], p0v)
    pltpu.sync_copy(pos_hbm.at[1, pl.ds(base, TOK_W)], p1v)
    pltpu.sync_copy(x_hbm.at[pl.ds(base, TOK_W)], xbuf)
    c0 = pltpu.async_copy(xbuf, xs_hbm.at[p0v], sem)
    c1 = pltpu.async_copy(xbuf, xs_hbm.at[p1v], sem)
    c0.wait()
    c1.wait()


def _dispatch(xbf, pos):
    mesh = plsc.VectorSubcoreMesh(core_axis_name="c", subcore_axis_name="s")
    f = functools.partial(
        pl.kernel,
        out_type=jax.ShapeDtypeStruct((PMAX, HALF), jnp.int32),
        mesh=mesh,
        scratch_types=[
            pltpu.VMEM((TOK_W,), jnp.int32),
            pltpu.VMEM((TOK_W,), jnp.int32),
            pltpu.VMEM((TOK_W, HALF), jnp.int32),
            pltpu.SemaphoreType.DMA,
        ],
    )(_dispatch_body)
    return f(xbf, pos)


def _experts_body(blk_ref, xs_ref, W1_ref, W2_ref, W3_ref, ys_ref):
    b = pl.program_id(0)

    @pl.when(b < blk_ref[120])
    def _():
        xv = _unpack_bf16(xs_ref[...])
        h = jax.nn.silu(jnp.dot(xv, W1_ref[0], preferred_element_type=jnp.float32))
        g = jnp.dot(xv, W3_ref[0], preferred_element_type=jnp.float32)
        y = jnp.dot(h * g, W2_ref[0], preferred_element_type=jnp.float32)
        ys_ref[...] = _pack_bf16(y)


def _experts(xs, blk_exp, W1, W2, W3):
    grid_spec = pltpu.PrefetchScalarGridSpec(
        num_scalar_prefetch=1,
        grid=(NBLK,),
        in_specs=[
            pl.BlockSpec((R, HALF), lambda b, blk: (b, 0)),
            pl.BlockSpec((1, DIM, HID), lambda b, blk: (blk[b], 0, 0)),
            pl.BlockSpec((1, HID, DIM), lambda b, blk: (blk[b], 0, 0)),
            pl.BlockSpec((1, DIM, HID), lambda b, blk: (blk[b], 0, 0)),
        ],
        out_specs=pl.BlockSpec((R, HALF), lambda b, blk: (b, 0)),
    )
    return pl.pallas_call(
        _experts_body,
        grid_spec=grid_spec,
        out_shape=jax.ShapeDtypeStruct((PMAX, HALF), jnp.int32),
    )(blk_exp, xs, W1, W2, W3)


def _gather_body(ys_hbm, pos_hbm, y0_hbm, y1_hbm, p0v, p1v, t0, t1, sem):
    wid = lax.axis_index("s") * 2 + lax.axis_index("c")
    for c in range(TOK_W // CH):
        base = wid * TOK_W + c * CH
        pltpu.sync_copy(pos_hbm.at[0, pl.ds(base, CH)], p0v)
        pltpu.sync_copy(pos_hbm.at[1, pl.ds(base, CH)], p1v)
        g0 = pltpu.async_copy(ys_hbm.at[p0v], t0, sem)
        g1 = pltpu.async_copy(ys_hbm.at[p1v], t1, sem)
        g0.wait()
        g1.wait()
        pltpu.sync_copy(t0, y0_hbm.at[pl.ds(base, CH)])
        pltpu.sync_copy(t1, y1_hbm.at[pl.ds(base, CH)])


def _gather(ys, pos):
    mesh = plsc.VectorSubcoreMesh(core_axis_name="c", subcore_axis_name="s")
    f = functools.partial(
        pl.kernel,
        out_type=[
            jax.ShapeDtypeStruct((T, HALF), jnp.int32),
            jax.ShapeDtypeStruct((T, HALF), jnp.int32),
        ],
        mesh=mesh,
        scratch_types=[
            pltpu.VMEM((CH,), jnp.int32),
            pltpu.VMEM((CH,), jnp.int32),
            pltpu.VMEM((CH, HALF), jnp.int32),
            pltpu.VMEM((CH, HALF), jnp.int32),
            pltpu.SemaphoreType.DMA,
        ],
    )(_gather_body)
    return f(ys, pos)


def _final_body(shared_ref, y0_ref, y1_ref, w01_ref, out_ref):
    w0 = w01_ref[0, :, :1]
    w1 = w01_ref[1, :, :1]
    y0 = _unpack_bf16(y0_ref[...])
    y1 = _unpack_bf16(y1_ref[...])
    out_ref[...] = shared_ref[...] + w0 * y0 + w1 * y1


def _final(shared, y0, y1, w01):
    return pl.pallas_call(
        _final_body,
        grid=(NTB,),
        in_specs=[
            pl.BlockSpec((TBLK, DIM), lambda tb: (tb, 0)),
            pl.BlockSpec((TBLK, HALF), lambda tb: (tb, 0)),
            pl.BlockSpec((TBLK, HALF), lambda tb: (tb, 0)),
            pl.BlockSpec((2, TBLK, 16), lambda tb: (0, tb, 0)),
        ],
        out_specs=pl.BlockSpec((TBLK, DIM), lambda tb: (tb, 0)),
        out_shape=jax.ShapeDtypeStruct((T, DIM), jnp.float32),
    )(shared, y0, y1, w01)


def kernel(x, w1s, w2s, w3s, W1, W2, W3, Wr):
    x2 = x.reshape(T, DIM)
    wr_pad = jnp.pad(Wr, ((0, 0), (0, 128 - E)))
    w01, pos, blk3, xbf = _router_meta(x2, wr_pad)
    blk_exp = blk3.reshape(128)
    xs = _dispatch(xbf, pos)
    shared = _shared_expert(x2, w1s, w3s, w2s)
    ys = _experts(xs, blk_exp, W1, W2, W3)
    y0, y1 = _gather(ys, pos)
    out = _final(shared, y0, y1, w01)
    return out.reshape(x.shape)


# speedup vs baseline: 1.1618x; 1.1618x over previous
import functools

import jax
import jax.numpy as jnp
from jax import lax
from jax.experimental import pallas as pl
from jax.experimental.pallas import tpu as pltpu
from jax.experimental.pallas import tpu_sc as plsc

T, DIM, E, HID = 2048, 768, 8, 1024
TBLK = 512
NTB = T // TBLK
R = 512
NBLK = T * 2 // R + E
PMAX = NBLK * R

NW = 32
TOK_W = T // NW
CH = 32


HALF = DIM // 2


def _pack_bf16(v):
    u = jax.lax.bitcast_convert_type(v, jnp.uint32)

    def rne(w):
        return (w + jnp.uint32(0x7FFF) + ((w >> 16) & jnp.uint32(1))) & jnp.uint32(0xFFFF0000)

    packed = rne(u[..., HALF:]) | (rne(u[..., :HALF]) >> 16)
    return jax.lax.bitcast_convert_type(packed, jnp.int32)


def _unpack_bf16(p):
    u = jax.lax.bitcast_convert_type(p, jnp.uint32)
    lo = jax.lax.bitcast_convert_type(u << 16, jnp.float32)
    hi = jax.lax.bitcast_convert_type(u & jnp.uint32(0xFFFF0000), jnp.float32)
    return jnp.concatenate([lo, hi], axis=-1)


def _shift_down(a, s):
    return jnp.concatenate([jnp.zeros((s,) + a.shape[1:], a.dtype), a[:-s]], axis=0)


def _shift_right_lanes(a, s):
    return jnp.concatenate([jnp.zeros(a.shape[:-1] + (s,), a.dtype), a[..., :-s]], axis=-1)


def _rm_body(x_ref, wr_ref,
             w01_ref, pos_ref, blk_ref, xbf_ref,
             sidx, hist, carry):
    ph = pl.program_id(0)
    tb = pl.program_id(1)
    lane = jax.lax.broadcasted_iota(jnp.int32, (TBLK, 128), 1)
    lane1 = jax.lax.broadcasted_iota(jnp.int32, (1, 128), 1)
    row = jax.lax.broadcasted_iota(jnp.int32, (8, TBLK), 0)

    @pl.when(ph == 0)
    def _():
        @pl.when(tb == 0)
        def _():
            hist[...] = jnp.zeros_like(hist)
            carry[...] = jnp.zeros_like(carry)

        xbf_ref[...] = _pack_bf16(x_ref[...])
        logits = jnp.dot(x_ref[...], wr_ref[...], preferred_element_type=jnp.float32)
        neg = jnp.float32(-1e30)
        l = jnp.where(lane < E, logits, neg)
        m0 = jnp.max(l, axis=-1, keepdims=True)
        i0 = jnp.min(jnp.where(l == m0, lane, jnp.int32(10**9)), axis=-1, keepdims=True)
        l2 = jnp.where(lane == i0, neg, l)
        m1 = jnp.max(l2, axis=-1, keepdims=True)
        i1 = jnp.min(jnp.where(l2 == m1, lane, jnp.int32(10**9)), axis=-1, keepdims=True)
        w0 = jax.nn.sigmoid(m0 - m1)
        i0r = jnp.broadcast_to(i0[:, 0][None, :], (8, TBLK))
        i1r = jnp.broadcast_to(i1[:, 0][None, :], (8, TBLK))
        packed = jnp.where(row == 0, i0r, jnp.where(row == 1, i1r, 0))
        sidx[:, pl.ds(tb * TBLK, TBLK)] = packed
        w0b = jnp.broadcast_to(w0, (TBLK, 16))
        w01_ref[...] = jnp.concatenate([w0b[None], (1.0 - w0b)[None]], axis=0)
        cnt = (lane == i0).astype(jnp.int32) + (lane == i1).astype(jnp.int32)
        hist[...] += jnp.sum(cnt, axis=0, keepdims=True)

    @pl.when(ph == 1)
    def _():
        tot = hist[...]
        sizes = ((tot + (R - 1)) // R) * R
        sizes = jnp.where(lane1 < E, sizes, 0)
        incl = sizes
        for s in (1, 2, 4):
            incl = incl + _shift_right_lanes(incl, s)
        poff = incl - sizes

        @pl.when(tb == 0)
        def _():
            pb = poff // R
            acc = jnp.full((1, 128), -1, jnp.int32)
            for e in range(E):
                pe = jnp.sum(jnp.where(lane1 == e, pb, 0))
                acc = acc + (lane1 >= pe).astype(jnp.int32)
            nused = jnp.sum(jnp.where(lane1 < E, sizes, 0)) // R
            blk_ref[...] = jnp.where(lane1 == 120, nused, acc)[None]

        i0 = sidx[0, pl.ds(tb * TBLK, TBLK)][:, None]
        i1 = sidx[1, pl.ds(tb * TBLK, TBLK)][:, None]
        cnt = (lane == i0).astype(jnp.int32) + (lane == i1).astype(jnp.int32)
        incl = cnt
        s = 1
        while s < TBLK:
            incl = incl + _shift_down(incl, s)
            s *= 2
        ec = incl - cnt
        base = poff + carry[...] + ec
        pos0 = jnp.sum(jnp.where(lane == i0, base, 0), axis=-1)
        pos1 = jnp.sum(jnp.where(lane == i1, base, 0), axis=-1)
        carry[...] += jnp.sum(cnt, axis=0, keepdims=True)
        p0r = jnp.broadcast_to(pos0[None, :], (8, TBLK))
        p1r = jnp.broadcast_to(pos1[None, :], (8, TBLK))
        pos_ref[...] = jnp.where(row == 0, p0r, jnp.where(row == 1, p1r, 0))


def _router_meta(x2, wr_pad):
    return pl.pallas_call(
        _rm_body,
        grid=(2, NTB),
        in_specs=[
            pl.BlockSpec((TBLK, DIM), lambda p, tb: (tb, 0)),
            pl.BlockSpec((DIM, 128), lambda p, tb: (0, 0)),
        ],
        out_specs=[
            pl.BlockSpec((2, TBLK, 16), lambda p, tb: (0, jnp.where(p == 0, tb, NTB - 1), 0)),
            pl.BlockSpec((8, TBLK), lambda p, tb: (0, jnp.where(p == 1, tb, 0))),
            pl.BlockSpec((1, 1, 128), lambda p, tb: (0, 0, 0)),
            pl.BlockSpec((TBLK, HALF), lambda p, tb: (jnp.where(p == 0, tb, NTB - 1), 0)),
        ],
        out_shape=[
            jax.ShapeDtypeStruct((2, T, 16), jnp.float32),
            jax.ShapeDtypeStruct((8, T), jnp.int32),
            jax.ShapeDtypeStruct((1, 1, 128), jnp.int32),
            jax.ShapeDtypeStruct((T, HALF), jnp.int32),
        ],
        scratch_shapes=[
            pltpu.VMEM((8, T), jnp.int32),
            pltpu.VMEM((1, 128), jnp.int32),
            pltpu.VMEM((1, 128), jnp.int32),
        ],
    )(x2, wr_pad)


def _shared_body(x_ref, w1s_ref, w3s_ref, w2s_ref, shared_ref):
    x = x_ref[...]
    h = jax.nn.silu(jnp.dot(x, w1s_ref[...], preferred_element_type=jnp.float32))
    g = jnp.dot(x, w3s_ref[...], preferred_element_type=jnp.float32)
    shared_ref[...] = jnp.dot(h * g, w2s_ref[...], preferred_element_type=jnp.float32)


def _shared_expert(x2, w1s, w3s, w2s):
    return pl.pallas_call(
        _shared_body,
        grid=(NTB,),
        in_specs=[
            pl.BlockSpec((TBLK, DIM), lambda tb: (tb, 0)),
            pl.BlockSpec((DIM, HID), lambda tb: (0, 0)),
            pl.BlockSpec((DIM, HID), lambda tb: (0, 0)),
            pl.BlockSpec((HID, DIM), lambda tb: (0, 0)),
        ],
        out_specs=pl.BlockSpec((TBLK, DIM), lambda tb: (tb, 0)),
        out_shape=jax.ShapeDtypeStruct((T, DIM), jnp.float32),
    )(x2, w1s, w3s, w2s)


def _dispatch_body(x_hbm, pos_hbm, xs_hbm, p0v, p1v, xbuf, sem):
    wid = lax.axis_index("s") * 2 + lax.axis_index("c")
    base = wid * TOK_W
    pltpu.sync_copy(pos_hbm.at[0, pl.ds(base, TOK_W)], p0v)
    pltpu.sync_copy(pos_hbm.at[1, pl.ds(base, TOK_W)], p1v)
    pltpu.sync_copy(x_hbm.at[pl.ds(base, TOK_W)], xbuf)
    c0 = pltpu.async_copy(xbuf, xs_hbm.at[p0v], sem)
    c1 = pltpu.async_copy(xbuf, xs_hbm.at[p1v], sem)
    c0.wait()
    c1.wait()


def _dispatch(xbf, pos):
    mesh = plsc.VectorSubcoreMesh(core_axis_name="c", subcore_axis_name="s")
    f = functools.partial(
        pl.kernel,
        out_type=jax.ShapeDtypeStruct((PMAX, HALF), jnp.int32),
        mesh=mesh,
        scratch_types=[
            pltpu.VMEM((TOK_W,), jnp.int32),
            pltpu.VMEM((TOK_W,), jnp.int32),
            pltpu.VMEM((TOK_W, HALF), jnp.int32),
            pltpu.SemaphoreType.DMA,
        ],
    )(_dispatch_body)
    return f(xbf, pos)


def _experts_body(blk_ref, xs_ref, W1_ref, W2_ref, W3_ref, ys_ref):
    b = pl.program_id(0)

    @pl.when(b < blk_ref[120])
    def _():
        xv = _unpack_bf16(xs_ref[...])
        h = jax.nn.silu(jnp.dot(xv, W1_ref[0], preferred_element_type=jnp.float32))
        g = jnp.dot(xv, W3_ref[0], preferred_element_type=jnp.float32)
        y = jnp.dot(h * g, W2_ref[0], preferred_element_type=jnp.float32)
        ys_ref[...] = _pack_bf16(y)


def _experts(xs, blk_exp, W1, W2, W3):
    grid_spec = pltpu.PrefetchScalarGridSpec(
        num_scalar_prefetch=1,
        grid=(NBLK,),
        in_specs=[
            pl.BlockSpec((R, HALF), lambda b, blk: (b, 0)),
            pl.BlockSpec((1, DIM, HID), lambda b, blk: (blk[b], 0, 0)),
            pl.BlockSpec((1, HID, DIM), lambda b, blk: (blk[b], 0, 0)),
            pl.BlockSpec((1, DIM, HID), lambda b, blk: (blk[b], 0, 0)),
        ],
        out_specs=pl.BlockSpec((R, HALF), lambda b, blk: (b, 0)),
    )
    return pl.pallas_call(
        _experts_body,
        grid_spec=grid_spec,
        out_shape=jax.ShapeDtypeStruct((PMAX, HALF), jnp.int32),
    )(blk_exp, xs, W1, W2, W3)


def _gather_body(ys_hbm, pos_hbm, y0_hbm, y1_hbm, p0v, p1v, t0, t1, sem):
    wid = lax.axis_index("s") * 2 + lax.axis_index("c")
    for c in range(TOK_W // CH):
        base = wid * TOK_W + c * CH
        pltpu.sync_copy(pos_hbm.at[0, pl.ds(base, CH)], p0v)
        pltpu.sync_copy(pos_hbm.at[1, pl.ds(base, CH)], p1v)
        g0 = pltpu.async_copy(ys_hbm.at[p0v], t0, sem)
        g1 = pltpu.async_copy(ys_hbm.at[p1v], t1, sem)
        g0.wait()
        g1.wait()
        pltpu.sync_copy(t0, y0_hbm.at[pl.ds(base, CH)])
        pltpu.sync_copy(t1, y1_hbm.at[pl.ds(base, CH)])


def _gather(ys, pos):
    mesh = plsc.VectorSubcoreMesh(core_axis_name="c", subcore_axis_name="s")
    f = functools.partial(
        pl.kernel,
        out_type=[
            jax.ShapeDtypeStruct((T, HALF), jnp.int32),
            jax.ShapeDtypeStruct((T, HALF), jnp.int32),
        ],
        mesh=mesh,
        scratch_types=[
            pltpu.VMEM((CH,), jnp.int32),
            pltpu.VMEM((CH,), jnp.int32),
            pltpu.VMEM((CH, HALF), jnp.int32),
            pltpu.VMEM((CH, HALF), jnp.int32),
            pltpu.SemaphoreType.DMA,
        ],
    )(_gather_body)
    return f(ys, pos)


def _final_body(shared_ref, y0_ref, y1_ref, w01_ref, out_ref):
    w0 = w01_ref[0, :, :1]
    w1 = w01_ref[1, :, :1]
    y0 = _unpack_bf16(y0_ref[...])
    y1 = _unpack_bf16(y1_ref[...])
    out_ref[...] = shared_ref[...] + w0 * y0 + w1 * y1


def _final(shared, y0, y1, w01):
    return pl.pallas_call(
        _final_body,
        grid=(NTB,),
        in_specs=[
            pl.BlockSpec((TBLK, DIM), lambda tb: (tb, 0)),
            pl.BlockSpec((TBLK, HALF), lambda tb: (tb, 0)),
            pl.BlockSpec((TBLK, HALF), lambda tb: (tb, 0)),
            pl.BlockSpec((2, TBLK, 16), lambda tb: (0, tb, 0)),
        ],
        out_specs=pl.BlockSpec((TBLK, DIM), lambda tb: (tb, 0)),
        out_shape=jax.ShapeDtypeStruct((T, DIM), jnp.float32),
    )(shared, y0, y1, w01)


def kernel(x, w1s, w2s, w3s, W1, W2, W3, Wr):
    x2 = x.reshape(T, DIM)
    wr_pad = jnp.pad(Wr, ((0, 0), (0, 128 - E)))
    w01, pos, blk3, xbf = _router_meta(x2, wr_pad)
    blk_exp = blk3.reshape(128)
    xs = _dispatch(xbf, pos)
    shared = _shared_expert(x2, w1s, w3s, w2s)
    ys = _experts(xs, blk_exp, W1, W2, W3)
    y0, y1 = _gather(ys, pos)
    out = _final(shared, y0, y1, w01)
    return out.reshape(x.shape)


# speedup vs baseline: 1.1887x; 1.0231x over previous
import functools

import jax
import jax.numpy as jnp
from jax import lax
from jax.experimental import pallas as pl
from jax.experimental.pallas import tpu as pltpu
from jax.experimental.pallas import tpu_sc as plsc

T, DIM, E, HID = 2048, 768, 8, 1024
TBLK = 1024
NTB = T // TBLK
R = 512
NBLK = T * 2 // R + E
PMAX = NBLK * R

NW = 32
TOK_W = T // NW
CH = 32


HALF = DIM // 2


def _pack_bf16(v):
    u = jax.lax.bitcast_convert_type(v, jnp.uint32)

    def rne(w):
        return (w + jnp.uint32(0x7FFF) + ((w >> 16) & jnp.uint32(1))) & jnp.uint32(0xFFFF0000)

    packed = rne(u[..., HALF:]) | (rne(u[..., :HALF]) >> 16)
    return jax.lax.bitcast_convert_type(packed, jnp.int32)


def _unpack_bf16(p):
    u = jax.lax.bitcast_convert_type(p, jnp.uint32)
    lo = jax.lax.bitcast_convert_type(u << 16, jnp.float32)
    hi = jax.lax.bitcast_convert_type(u & jnp.uint32(0xFFFF0000), jnp.float32)
    return jnp.concatenate([lo, hi], axis=-1)


def _shift_down(a, s):
    return jnp.concatenate([jnp.zeros((s,) + a.shape[1:], a.dtype), a[:-s]], axis=0)


def _shift_right_lanes(a, s):
    return jnp.concatenate([jnp.zeros(a.shape[:-1] + (s,), a.dtype), a[..., :-s]], axis=-1)


def _rm_body(x_ref, wr_ref,
             w01_ref, pos_ref, blk_ref, xbf_ref,
             sidx, hist, carry):
    ph = pl.program_id(0)
    tb = pl.program_id(1)
    lane = jax.lax.broadcasted_iota(jnp.int32, (TBLK, 128), 1)
    lane1 = jax.lax.broadcasted_iota(jnp.int32, (1, 128), 1)
    row = jax.lax.broadcasted_iota(jnp.int32, (8, TBLK), 0)

    @pl.when(ph == 0)
    def _():
        @pl.when(tb == 0)
        def _():
            hist[...] = jnp.zeros_like(hist)
            carry[...] = jnp.zeros_like(carry)

        xbf_ref[...] = _pack_bf16(x_ref[...])
        logits = jnp.dot(x_ref[...], wr_ref[...], preferred_element_type=jnp.float32)
        neg = jnp.float32(-1e30)
        l = jnp.where(lane < E, logits, neg)
        m0 = jnp.max(l, axis=-1, keepdims=True)
        i0 = jnp.min(jnp.where(l == m0, lane, jnp.int32(10**9)), axis=-1, keepdims=True)
        l2 = jnp.where(lane == i0, neg, l)
        m1 = jnp.max(l2, axis=-1, keepdims=True)
        i1 = jnp.min(jnp.where(l2 == m1, lane, jnp.int32(10**9)), axis=-1, keepdims=True)
        w0 = jax.nn.sigmoid(m0 - m1)
        i0r = jnp.broadcast_to(i0[:, 0][None, :], (8, TBLK))
        i1r = jnp.broadcast_to(i1[:, 0][None, :], (8, TBLK))
        packed = jnp.where(row == 0, i0r, jnp.where(row == 1, i1r, 0))
        sidx[:, pl.ds(tb * TBLK, TBLK)] = packed
        w0b = jnp.broadcast_to(w0, (TBLK, 16))
        w01_ref[...] = jnp.concatenate([w0b[None], (1.0 - w0b)[None]], axis=0)
        cnt = (lane == i0).astype(jnp.int32) + (lane == i1).astype(jnp.int32)
        hist[...] += jnp.sum(cnt, axis=0, keepdims=True)

    @pl.when(ph == 1)
    def _():
        tot = hist[...]
        sizes = ((tot + (R - 1)) // R) * R
        sizes = jnp.where(lane1 < E, sizes, 0)
        incl = sizes
        for s in (1, 2, 4):
            incl = incl + _shift_right_lanes(incl, s)
        poff = incl - sizes

        @pl.when(tb == 0)
        def _():
            pb = poff // R
            acc = jnp.full((1, 128), -1, jnp.int32)
            for e in range(E):
                pe = jnp.sum(jnp.where(lane1 == e, pb, 0))
                acc = acc + (lane1 >= pe).astype(jnp.int32)
            nused = jnp.sum(jnp.where(lane1 < E, sizes, 0)) // R
            blk_ref[...] = jnp.where(lane1 == 120, nused, acc)[None]

        i0 = sidx[0, pl.ds(tb * TBLK, TBLK)][:, None]
        i1 = sidx[1, pl.ds(tb * TBLK, TBLK)][:, None]
        cnt = (lane == i0).astype(jnp.int32) + (lane == i1).astype(jnp.int32)
        incl = cnt
        s = 1
        while s < TBLK:
            incl = incl + _shift_down(incl, s)
            s *= 2
        ec = incl - cnt
        base = poff + carry[...] + ec
        pos0 = jnp.sum(jnp.where(lane == i0, base, 0), axis=-1)
        pos1 = jnp.sum(jnp.where(lane == i1, base, 0), axis=-1)
        carry[...] += jnp.sum(cnt, axis=0, keepdims=True)
        p0r = jnp.broadcast_to(pos0[None, :], (8, TBLK))
        p1r = jnp.broadcast_to(pos1[None, :], (8, TBLK))
        pos_ref[...] = jnp.where(row == 0, p0r, jnp.where(row == 1, p1r, 0))


def _router_meta(x2, wr_pad):
    return pl.pallas_call(
        _rm_body,
        grid=(2, NTB),
        in_specs=[
            pl.BlockSpec((TBLK, DIM), lambda p, tb: (tb, 0)),
            pl.BlockSpec((DIM, 128), lambda p, tb: (0, 0)),
        ],
        out_specs=[
            pl.BlockSpec((2, TBLK, 16), lambda p, tb: (0, jnp.where(p == 0, tb, NTB - 1), 0)),
            pl.BlockSpec((8, TBLK), lambda p, tb: (0, jnp.where(p == 1, tb, 0))),
            pl.BlockSpec((1, 1, 128), lambda p, tb: (0, 0, 0)),
            pl.BlockSpec((TBLK, HALF), lambda p, tb: (jnp.where(p == 0, tb, NTB - 1), 0)),
        ],
        out_shape=[
            jax.ShapeDtypeStruct((2, T, 16), jnp.float32),
            jax.ShapeDtypeStruct((8, T), jnp.int32),
            jax.ShapeDtypeStruct((1, 1, 128), jnp.int32),
            jax.ShapeDtypeStruct((T, HALF), jnp.int32),
        ],
        scratch_shapes=[
            pltpu.VMEM((8, T), jnp.int32),
            pltpu.VMEM((1, 128), jnp.int32),
            pltpu.VMEM((1, 128), jnp.int32),
        ],
    )(x2, wr_pad)


def _shared_body(x_ref, w1s_ref, w3s_ref, w2s_ref, shared_ref):
    x = x_ref[...]
    h = jax.nn.silu(jnp.dot(x, w1s_ref[...], preferred_element_type=jnp.float32))
    g = jnp.dot(x, w3s_ref[...], preferred_element_type=jnp.float32)
    shared_ref[...] = jnp.dot(h * g, w2s_ref[...], preferred_element_type=jnp.float32)


def _shared_expert(x2, w1s, w3s, w2s):
    return pl.pallas_call(
        _shared_body,
        grid=(NTB,),
        in_specs=[
            pl.BlockSpec((TBLK, DIM), lambda tb: (tb, 0)),
            pl.BlockSpec((DIM, HID), lambda tb: (0, 0)),
            pl.BlockSpec((DIM, HID), lambda tb: (0, 0)),
            pl.BlockSpec((HID, DIM), lambda tb: (0, 0)),
        ],
        out_specs=pl.BlockSpec((TBLK, DIM), lambda tb: (tb, 0)),
        out_shape=jax.ShapeDtypeStruct((T, DIM), jnp.float32),
    )(x2, w1s, w3s, w2s)


def _dispatch_body(x_hbm, pos_hbm, xs_hbm, p0v, p1v, xbuf, sem):
    wid = lax.axis_index("s") * 2 + lax.axis_index("c")
    base = wid * TOK_W
    pltpu.sync_copy(pos_hbm.at[0, pl.ds(base, TOK_W)], p0v)
    pltpu.sync_copy(pos_hbm.at[1, pl.ds(base, TOK_W)], p1v)
    pltpu.sync_copy(x_hbm.at[pl.ds(base, TOK_W)], xbuf)
    c0 = pltpu.async_copy(xbuf, xs_hbm.at[p0v], sem)
    c1 = pltpu.async_copy(xbuf, xs_hbm.at[p1v], sem)
    c0.wait()
    c1.wait()


def _dispatch(xbf, pos):
    mesh = plsc.VectorSubcoreMesh(core_axis_name="c", subcore_axis_name="s")
    f = functools.partial(
        pl.kernel,
        out_type=jax.ShapeDtypeStruct((PMAX, HALF), jnp.int32),
        mesh=mesh,
        scratch_types=[
            pltpu.VMEM((TOK_W,), jnp.int32),
            pltpu.VMEM((TOK_W,), jnp.int32),
            pltpu.VMEM((TOK_W, HALF), jnp.int32),
            pltpu.SemaphoreType.DMA,
        ],
    )(_dispatch_body)
    return f(xbf, pos)


def _experts_body(blk_ref, xs_ref, W1_ref, W2_ref, W3_ref, ys_ref):
    b = pl.program_id(0)

    @pl.when(b < blk_ref[120])
    def _():
        xv = _unpack_bf16(xs_ref[...])
        h = jax.nn.silu(jnp.dot(xv, W1_ref[0], preferred_element_type=jnp.float32))
        g = jnp.dot(xv, W3_ref[0], preferred_element_type=jnp.float32)
        y = jnp.dot(h * g, W2_ref[0], preferred_element_type=jnp.float32)
        ys_ref[...] = _pack_bf16(y)


def _experts(xs, blk_exp, W1, W2, W3):
    grid_spec = pltpu.PrefetchScalarGridSpec(
        num_scalar_prefetch=1,
        grid=(NBLK,),
        in_specs=[
            pl.BlockSpec((R, HALF), lambda b, blk: (b, 0)),
            pl.BlockSpec((1, DIM, HID), lambda b, blk: (blk[b], 0, 0)),
            pl.BlockSpec((1, HID, DIM), lambda b, blk: (blk[b], 0, 0)),
            pl.BlockSpec((1, DIM, HID), lambda b, blk: (blk[b], 0, 0)),
        ],
        out_specs=pl.BlockSpec((R, HALF), lambda b, blk: (b, 0)),
    )
    return pl.pallas_call(
        _experts_body,
        grid_spec=grid_spec,
        out_shape=jax.ShapeDtypeStruct((PMAX, HALF), jnp.int32),
    )(blk_exp, xs, W1, W2, W3)


def _gather_body(ys_hbm, pos_hbm, y0_hbm, y1_hbm, p0v, p1v, t0, t1, sem):
    wid = lax.axis_index("s") * 2 + lax.axis_index("c")
    for c in range(TOK_W // CH):
        base = wid * TOK_W + c * CH
        pltpu.sync_copy(pos_hbm.at[0, pl.ds(base, CH)], p0v)
        pltpu.sync_copy(pos_hbm.at[1, pl.ds(base, CH)], p1v)
        g0 = pltpu.async_copy(ys_hbm.at[p0v], t0, sem)
        g1 = pltpu.async_copy(ys_hbm.at[p1v], t1, sem)
        g0.wait()
        g1.wait()
        pltpu.sync_copy(t0, y0_hbm.at[pl.ds(base, CH)])
        pltpu.sync_copy(t1, y1_hbm.at[pl.ds(base, CH)])


def _gather(ys, pos):
    mesh = plsc.VectorSubcoreMesh(core_axis_name="c", subcore_axis_name="s")
    f = functools.partial(
        pl.kernel,
        out_type=[
            jax.ShapeDtypeStruct((T, HALF), jnp.int32),
            jax.ShapeDtypeStruct((T, HALF), jnp.int32),
        ],
        mesh=mesh,
        scratch_types=[
            pltpu.VMEM((CH,), jnp.int32),
            pltpu.VMEM((CH,), jnp.int32),
            pltpu.VMEM((CH, HALF), jnp.int32),
            pltpu.VMEM((CH, HALF), jnp.int32),
            pltpu.SemaphoreType.DMA,
        ],
    )(_gather_body)
    return f(ys, pos)


def _final_body(shared_ref, y0_ref, y1_ref, w01_ref, out_ref):
    w0 = w01_ref[0, :, :1]
    w1 = w01_ref[1, :, :1]
    y0 = _unpack_bf16(y0_ref[...])
    y1 = _unpack_bf16(y1_ref[...])
    out_ref[...] = shared_ref[...] + w0 * y0 + w1 * y1


def _final(shared, y0, y1, w01):
    return pl.pallas_call(
        _final_body,
        grid=(NTB,),
        in_specs=[
            pl.BlockSpec((TBLK, DIM), lambda tb: (tb, 0)),
            pl.BlockSpec((TBLK, HALF), lambda tb: (tb, 0)),
            pl.BlockSpec((TBLK, HALF), lambda tb: (tb, 0)),
            pl.BlockSpec((2, TBLK, 16), lambda tb: (0, tb, 0)),
        ],
        out_specs=pl.BlockSpec((TBLK, DIM), lambda tb: (tb, 0)),
        out_shape=jax.ShapeDtypeStruct((T, DIM), jnp.float32),
    )(shared, y0, y1, w01)


def kernel(x, w1s, w2s, w3s, W1, W2, W3, Wr):
    x2 = x.reshape(T, DIM)
    wr_pad = jnp.pad(Wr, ((0, 0), (0, 128 - E)))
    w01, pos, blk3, xbf = _router_meta(x2, wr_pad)
    blk_exp = blk3.reshape(128)
    xs = _dispatch(xbf, pos)
    shared = _shared_expert(x2, w1s, w3s, w2s)
    ys = _experts(xs, blk_exp, W1, W2, W3)
    y0, y1 = _gather(ys, pos)
    out = _final(shared, y0, y1, w01)
    return out.reshape(x.shape)
